# Initial kernel scaffold; baseline (speedup 1.0000x reference)
import functools
import math

import jax
import jax.numpy as jnp
from jax import lax
from jax.experimental import pallas as pl
from jax.experimental.pallas import tpu as pltpu
from jax.experimental.pallas import tpu_sc as plsc

N = 10000
E = 320000
D = 128
NC = 2
NS = 16
NW = NC * NS
CHUNK = 128
NCH = 80
E_PAD = NW * NCH * CHUNK
N_ACC = 10240
TPW = N_ACC // NS
RB = 1250

ALPHA = 0.1
THETA1 = math.log(2.0)
THETA2 = math.log(1.5)

_MESH = plsc.VectorSubcoreMesh(core_axis_name="c", subcore_axis_name="s")


def _deg_partials(colh, ones16, zeros16):
    @functools.partial(
        pl.kernel,
        out_type=jax.ShapeDtypeStruct((NC, N_ACC, 16), jnp.float32),
        mesh=_MESH,
        scratch_types=[
            pltpu.VMEM((NCH, CHUNK), jnp.int32),
            pltpu.VMEM((CHUNK, 16), jnp.float32),
            pltpu.VMEM_SHARED((N_ACC, 16), jnp.float32),
        ],
    )
    def k(colh_hbm, ones_hbm, zeros_hbm, out_hbm, colv, onesv, acc):
        c = lax.axis_index("c")
        s = lax.axis_index("s")
        w = c * NS + s
        pltpu.sync_copy(colh_hbm.at[w], colv)
        pltpu.sync_copy(ones_hbm, onesv)
        pltpu.sync_copy(zeros_hbm, acc.at[pl.ds(s * TPW, TPW)])
        plsc.subcore_barrier()

        @pl.loop(0, NCH)
        def _(j):
            pltpu.sync_copy(onesv, acc.at[colv.at[j]], add=True)

        plsc.subcore_barrier()
        pltpu.sync_copy(acc.at[pl.ds(s * TPW, TPW)],
                        out_hbm.at[c].at[pl.ds(s * TPW, TPW)])

    return k(colh, ones16, zeros16)


def _spmm_partials(g, colg, rows, zeros128):
    @functools.partial(
        pl.kernel,
        out_type=jax.ShapeDtypeStruct((NC, N_ACC, D), jnp.float32),
        mesh=_MESH,
        scratch_types=[
            pltpu.VMEM((NCH, CHUNK), jnp.int32),
            pltpu.VMEM((NCH, CHUNK), jnp.int32),
            pltpu.VMEM((CHUNK, D), jnp.float32),
            pltpu.VMEM_SHARED((N_ACC, D), jnp.float32),
        ],
    )
    def k(g_hbm, colg_hbm, rows_hbm, zeros_hbm, out_hbm, colv, rowv, data, acc):
        c = lax.axis_index("c")
        s = lax.axis_index("s")
        w = c * NS + s
        pltpu.sync_copy(colg_hbm.at[w], colv)
        pltpu.sync_copy(rows_hbm.at[w], rowv)
        pltpu.sync_copy(zeros_hbm, acc.at[pl.ds(s * TPW, TPW)])
        plsc.subcore_barrier()

        @pl.loop(0, NCH)
        def _(j):
            pltpu.sync_copy(g_hbm.at[colv.at[j]], data)
            pltpu.sync_copy(data, acc.at[rowv.at[j]], add=True)

        plsc.subcore_barrier()
        pltpu.sync_copy(acc.at[pl.ds(s * TPW, TPW)],
                        out_hbm.at[c].at[pl.ds(s * TPW, TPW)])

    return k(g, colg, rows, zeros128)


def _fc0(x, fc0_w, fc0_b):
    def body(x_ref, w_ref, b_ref, o_ref):
        o_ref[...] = jnp.maximum(
            jnp.dot(x_ref[...], w_ref[...],
                    preferred_element_type=jnp.float32,
                    precision=lax.Precision.HIGHEST) + b_ref[...], 0.0)

    return pl.pallas_call(
        body,
        grid=(N // RB,),
        in_specs=[
            pl.BlockSpec((RB, D), lambda i: (i, 0)),
            pl.BlockSpec((D, D), lambda i: (0, 0)),
            pl.BlockSpec((1, D), lambda i: (0, 0)),
        ],
        out_specs=pl.BlockSpec((RB, D), lambda i: (i, 0)),
        out_shape=jax.ShapeDtypeStruct((N, D), jnp.float32),
    )(x, fc0_w, fc0_b.reshape(1, D))


def _prep(degp, h):
    def body(degp_ref, h_ref, dis_ref, g_ref):
        d = degp_ref[...]
        deg = d[0][:, :1] + d[1][:, :1] + 1.0
        dis = lax.rsqrt(deg)
        dis_b = jnp.broadcast_to(dis, (RB, D))
        dis_ref[...] = dis_b
        g_ref[...] = dis_b * h_ref[...]

    return pl.pallas_call(
        body,
        grid=(N // RB,),
        in_specs=[
            pl.BlockSpec((NC, RB, 16), lambda i: (0, i, 0)),
            pl.BlockSpec((RB, D), lambda i: (i, 0)),
        ],
        out_specs=[
            pl.BlockSpec((RB, D), lambda i: (i, 0)),
            pl.BlockSpec((RB, D), lambda i: (i, 0)),
        ],
        out_shape=[
            jax.ShapeDtypeStruct((N, D), jnp.float32),
            jax.ShapeDtypeStruct((N, D), jnp.float32),
        ],
    )(degp, h)


def _layer(sp, g, h0, dis, w, theta):
    def body(sp_ref, g_ref, h0_ref, dis_ref, w_ref, o_ref):
        s = sp_ref[...]
        dis_b = dis_ref[...]
        hi = dis_b * (s[0] + s[1] + g_ref[...])
        sup = (1.0 - ALPHA) * hi + ALPHA * h0_ref[...]
        hn = jnp.maximum(
            theta * jnp.dot(sup, w_ref[...],
                            preferred_element_type=jnp.float32,
                            precision=lax.Precision.HIGHEST)
            + (1.0 - theta) * sup, 0.0)
        o_ref[...] = dis_b * hn

    return pl.pallas_call(
        body,
        grid=(N // RB,),
        in_specs=[
            pl.BlockSpec((NC, RB, D), lambda i: (0, i, 0)),
            pl.BlockSpec((RB, D), lambda i: (i, 0)),
            pl.BlockSpec((RB, D), lambda i: (i, 0)),
            pl.BlockSpec((RB, D), lambda i: (i, 0)),
            pl.BlockSpec((D, D), lambda i: (0, 0)),
        ],
        out_specs=pl.BlockSpec((RB, D), lambda i: (i, 0)),
        out_shape=jax.ShapeDtypeStruct((N, D), jnp.float32),
    )(sp, g, h0, dis, w)


def _final(sp, g1, h0, dis, w2, fc1_w, fc1_b):
    def body(sp_ref, g_ref, h0_ref, dis_ref, w_ref, fw_ref, fb_ref, o_ref):
        s = sp_ref[...]
        hi = dis_ref[...] * (s[0] + s[1] + g_ref[...])
        sup = (1.0 - ALPHA) * hi + ALPHA * h0_ref[...]
        h2 = jnp.maximum(
            THETA2 * jnp.dot(sup, w_ref[...],
                             preferred_element_type=jnp.float32,
                             precision=lax.Precision.HIGHEST)
            + (1.0 - THETA2) * sup, 0.0)
        o = jnp.dot(h2, fw_ref[...],
                    preferred_element_type=jnp.float32,
                    precision=lax.Precision.HIGHEST) + fb_ref[...]
        m = jnp.max(o, axis=1, keepdims=True)
        o_ref[...] = o - m - jnp.log(
            jnp.sum(jnp.exp(o - m), axis=1, keepdims=True))

    return pl.pallas_call(
        body,
        grid=(N // RB,),
        in_specs=[
            pl.BlockSpec((NC, RB, D), lambda i: (0, i, 0)),
            pl.BlockSpec((RB, D), lambda i: (i, 0)),
            pl.BlockSpec((RB, D), lambda i: (i, 0)),
            pl.BlockSpec((RB, D), lambda i: (i, 0)),
            pl.BlockSpec((D, D), lambda i: (0, 0)),
            pl.BlockSpec((D, D), lambda i: (0, 0)),
            pl.BlockSpec((1, D), lambda i: (0, 0)),
        ],
        out_specs=pl.BlockSpec((RB, D), lambda i: (i, 0)),
        out_shape=jax.ShapeDtypeStruct((N, D), jnp.float32),
    )(sp, g1, h0, dis, w2, fc1_w, fc1_b.reshape(1, D))


def kernel(x, edge_index, fc0_w, fc0_b, w1, w2, fc1_w, fc1_b):
    row = edge_index[0]
    col = edge_index[1]
    pad = E_PAD - E
    colh = jnp.concatenate([col, jnp.full((pad,), N, jnp.int32)])
    colg = jnp.concatenate([col, jnp.zeros((pad,), jnp.int32)])
    rows = jnp.concatenate([row, jnp.full((pad,), N, jnp.int32)])
    colh = colh.reshape(NW, NCH, CHUNK)
    colg = colg.reshape(NW, NCH, CHUNK)
    rows = rows.reshape(NW, NCH, CHUNK)
    ones16 = jnp.ones((CHUNK, 16), jnp.float32)
    zeros16 = jnp.zeros((TPW, 16), jnp.float32)
    zeros128 = jnp.zeros((TPW, D), jnp.float32)

    degp = _deg_partials(colh, ones16, zeros16)
    h0 = _fc0(x, fc0_w, fc0_b)
    dis, g0 = _prep(degp, h0)
    sp1 = _spmm_partials(g0, colg, rows, zeros128)
    g1 = _layer(sp1, g0, h0, dis, w1, THETA1)
    sp2 = _spmm_partials(g1, colg, rows, zeros128)
    return _final(sp2, g1, h0, dis, w2, fc1_w, fc1_b)


# speedup vs baseline: 8.1086x; 8.1086x over previous
import functools
import math

import jax
import jax.numpy as jnp
from jax import lax
from jax.experimental import pallas as pl
from jax.experimental.pallas import tpu as pltpu
from jax.experimental.pallas import tpu_sc as plsc

N = 10000
E = 320000
D = 128
NC = 2
NS = 16
NW = NC * NS
CHUNK = 128
NCH = 80
E_PAD = NW * NCH * CHUNK
N_ACC = 10240
TPW = N_ACC // NS
RB = 2000

ALPHA = 0.1
THETA1 = math.log(2.0)
THETA2 = math.log(1.5)

_MESH = plsc.VectorSubcoreMesh(core_axis_name="c", subcore_axis_name="s")


def _deg_partials(colh, ones128, zeros128):
    @functools.partial(
        pl.kernel,
        out_type=jax.ShapeDtypeStruct((NC, N_ACC, D), jnp.float32),
        mesh=_MESH,
        scratch_types=[
            pltpu.VMEM((NCH, CHUNK), jnp.int32),
            pltpu.VMEM((CHUNK, D), jnp.float32),
            pltpu.VMEM_SHARED((N_ACC, D), jnp.float32),
        ],
    )
    def k(colh_hbm, ones_hbm, zeros_hbm, out_hbm, colv, onesv, acc):
        c = lax.axis_index("c")
        s = lax.axis_index("s")
        w = c * NS + s
        pltpu.sync_copy(colh_hbm.at[w], colv)
        pltpu.sync_copy(ones_hbm, onesv)
        pltpu.sync_copy(zeros_hbm, acc.at[pl.ds(s * TPW, TPW)])
        plsc.subcore_barrier()

        @pl.loop(0, NCH)
        def _(j):
            pltpu.sync_copy(onesv, acc.at[colv.at[j]], add=True)

        plsc.subcore_barrier()
        pltpu.sync_copy(acc.at[pl.ds(s * TPW, TPW)],
                        out_hbm.at[c].at[pl.ds(s * TPW, TPW)])

    return k(colh, ones128, zeros128)


def _spmm_partials(g, colg, rows, zeros128):
    @functools.partial(
        pl.kernel,
        out_type=jax.ShapeDtypeStruct((NC, N_ACC, D), jnp.float32),
        mesh=_MESH,
        scratch_types=[
            pltpu.VMEM((NCH, CHUNK), jnp.int32),
            pltpu.VMEM((NCH, CHUNK), jnp.int32),
            pltpu.VMEM((CHUNK, D), jnp.float32),
            pltpu.VMEM_SHARED((N_ACC, D), jnp.float32),
        ],
    )
    def k(g_hbm, colg_hbm, rows_hbm, zeros_hbm, out_hbm, colv, rowv, data, acc):
        c = lax.axis_index("c")
        s = lax.axis_index("s")
        w = c * NS + s
        pltpu.sync_copy(colg_hbm.at[w], colv)
        pltpu.sync_copy(rows_hbm.at[w], rowv)
        pltpu.sync_copy(zeros_hbm, acc.at[pl.ds(s * TPW, TPW)])
        plsc.subcore_barrier()

        @pl.loop(0, NCH)
        def _(j):
            pltpu.sync_copy(g_hbm.at[colv.at[j]], data)
            pltpu.sync_copy(data, acc.at[rowv.at[j]], add=True)

        plsc.subcore_barrier()
        pltpu.sync_copy(acc.at[pl.ds(s * TPW, TPW)],
                        out_hbm.at[c].at[pl.ds(s * TPW, TPW)])

    return k(g, colg, rows, zeros128)


def _fc0(x, fc0_w, fc0_b):
    def body(x_ref, w_ref, b_ref, o_ref):
        o_ref[...] = jnp.maximum(
            jnp.dot(x_ref[...], w_ref[...],
                    preferred_element_type=jnp.float32,
                    precision=lax.Precision.HIGHEST) + b_ref[...], 0.0)

    return pl.pallas_call(
        body,
        grid=(N // RB,),
        in_specs=[
            pl.BlockSpec((RB, D), lambda i: (i, 0)),
            pl.BlockSpec((D, D), lambda i: (0, 0)),
            pl.BlockSpec((1, D), lambda i: (0, 0)),
        ],
        out_specs=pl.BlockSpec((RB, D), lambda i: (i, 0)),
        out_shape=jax.ShapeDtypeStruct((N, D), jnp.float32),
    )(x, fc0_w, fc0_b.reshape(1, D))


def _prep(degp, h):
    def body(degp_ref, h_ref, dis_ref, g_ref):
        d = degp_ref[...]
        deg = d[0][:, :1] + d[1][:, :1] + 1.0
        dis = lax.rsqrt(deg)
        dis_b = jnp.broadcast_to(dis, (RB, D))
        dis_ref[...] = dis_b
        g_ref[...] = dis_b * h_ref[...]

    return pl.pallas_call(
        body,
        grid=(N // RB,),
        in_specs=[
            pl.BlockSpec((NC, RB, D), lambda i: (0, i, 0)),
            pl.BlockSpec((RB, D), lambda i: (i, 0)),
        ],
        out_specs=[
            pl.BlockSpec((RB, D), lambda i: (i, 0)),
            pl.BlockSpec((RB, D), lambda i: (i, 0)),
        ],
        out_shape=[
            jax.ShapeDtypeStruct((N, D), jnp.float32),
            jax.ShapeDtypeStruct((N, D), jnp.float32),
        ],
    )(degp, h)


def _layer(sp, g, h0, dis, w, theta):
    def body(sp_ref, g_ref, h0_ref, dis_ref, w_ref, o_ref):
        s = sp_ref[...]
        dis_b = dis_ref[...]
        hi = dis_b * (s[0] + s[1] + g_ref[...])
        sup = (1.0 - ALPHA) * hi + ALPHA * h0_ref[...]
        hn = jnp.maximum(
            theta * jnp.dot(sup, w_ref[...],
                            preferred_element_type=jnp.float32,
                            precision=lax.Precision.HIGHEST)
            + (1.0 - theta) * sup, 0.0)
        o_ref[...] = dis_b * hn

    return pl.pallas_call(
        body,
        grid=(N // RB,),
        in_specs=[
            pl.BlockSpec((NC, RB, D), lambda i: (0, i, 0)),
            pl.BlockSpec((RB, D), lambda i: (i, 0)),
            pl.BlockSpec((RB, D), lambda i: (i, 0)),
            pl.BlockSpec((RB, D), lambda i: (i, 0)),
            pl.BlockSpec((D, D), lambda i: (0, 0)),
        ],
        out_specs=pl.BlockSpec((RB, D), lambda i: (i, 0)),
        out_shape=jax.ShapeDtypeStruct((N, D), jnp.float32),
    )(sp, g, h0, dis, w)


def _final(sp, g1, h0, dis, w2, fc1_w, fc1_b):
    def body(sp_ref, g_ref, h0_ref, dis_ref, w_ref, fw_ref, fb_ref, o_ref):
        s = sp_ref[...]
        hi = dis_ref[...] * (s[0] + s[1] + g_ref[...])
        sup = (1.0 - ALPHA) * hi + ALPHA * h0_ref[...]
        h2 = jnp.maximum(
            THETA2 * jnp.dot(sup, w_ref[...],
                             preferred_element_type=jnp.float32,
                             precision=lax.Precision.HIGHEST)
            + (1.0 - THETA2) * sup, 0.0)
        o = jnp.dot(h2, fw_ref[...],
                    preferred_element_type=jnp.float32,
                    precision=lax.Precision.HIGHEST) + fb_ref[...]
        m = jnp.max(o, axis=1, keepdims=True)
        o_ref[...] = o - m - jnp.log(
            jnp.sum(jnp.exp(o - m), axis=1, keepdims=True))

    return pl.pallas_call(
        body,
        grid=(N // RB,),
        in_specs=[
            pl.BlockSpec((NC, RB, D), lambda i: (0, i, 0)),
            pl.BlockSpec((RB, D), lambda i: (i, 0)),
            pl.BlockSpec((RB, D), lambda i: (i, 0)),
            pl.BlockSpec((RB, D), lambda i: (i, 0)),
            pl.BlockSpec((D, D), lambda i: (0, 0)),
            pl.BlockSpec((D, D), lambda i: (0, 0)),
            pl.BlockSpec((1, D), lambda i: (0, 0)),
        ],
        out_specs=pl.BlockSpec((RB, D), lambda i: (i, 0)),
        out_shape=jax.ShapeDtypeStruct((N, D), jnp.float32),
    )(sp, g1, h0, dis, w2, fc1_w, fc1_b.reshape(1, D))


def kernel(x, edge_index, fc0_w, fc0_b, w1, w2, fc1_w, fc1_b):
    row = edge_index[0]
    col = edge_index[1]
    pad = E_PAD - E
    colh = jnp.concatenate([col, jnp.full((pad,), N, jnp.int32)])
    colg = jnp.concatenate([col, jnp.zeros((pad,), jnp.int32)])
    rows = jnp.concatenate([row, jnp.full((pad,), N, jnp.int32)])
    colh = colh.reshape(NW, NCH, CHUNK)
    colg = colg.reshape(NW, NCH, CHUNK)
    rows = rows.reshape(NW, NCH, CHUNK)
    ones128 = jnp.ones((CHUNK, D), jnp.float32)
    zeros128 = jnp.zeros((TPW, D), jnp.float32)

    degp = _deg_partials(colh, ones128, zeros128)
    h0 = _fc0(x, fc0_w, fc0_b)
    dis, g0 = _prep(degp, h0)
    sp1 = _spmm_partials(g0, colg, rows, zeros128)
    g1 = _layer(sp1, g0, h0, dis, w1, THETA1)
    sp2 = _spmm_partials(g1, colg, rows, zeros128)
    return _final(sp2, g1, h0, dis, w2, fc1_w, fc1_b)


# speedup vs baseline: 8.9121x; 1.0991x over previous
import functools
import math

import jax
import jax.numpy as jnp
from jax import lax
from jax.experimental import pallas as pl
from jax.experimental.pallas import tpu as pltpu
from jax.experimental.pallas import tpu_sc as plsc

N = 10000
E = 320000
D = 128
NC = 2
NS = 16
NW = NC * NS
CHUNK = 64
NCH = 160
PNCH = 40
NPH = 4
NB = 4
E_PAD = NW * NCH * CHUNK
N_ACC = 10112
TPW = N_ACC // NS
RB = 2000

ALPHA = 0.1
THETA1 = math.log(2.0)
THETA2 = math.log(1.5)

_MESH = plsc.VectorSubcoreMesh(core_axis_name="c", subcore_axis_name="s")


def _deg_partials(colh, ones128, zeros128):
    @functools.partial(
        pl.kernel,
        out_type=jax.ShapeDtypeStruct((NC, N_ACC, D), jnp.float32),
        mesh=_MESH,
        scratch_types=[
            pltpu.VMEM((NCH, CHUNK), jnp.int32),
            pltpu.VMEM((CHUNK, D), jnp.float32),
            pltpu.VMEM_SHARED((N_ACC, D), jnp.float32),
            pltpu.SemaphoreType.DMA,
        ],
    )
    def k(colh_hbm, ones_hbm, zeros_hbm, out_hbm, colv, onesv, acc, sem):
        c = lax.axis_index("c")
        s = lax.axis_index("s")
        w = c * NS + s
        pltpu.sync_copy(colh_hbm.at[w], colv)
        pltpu.sync_copy(ones_hbm, onesv)
        pltpu.sync_copy(zeros_hbm, acc.at[pl.ds(s * TPW, TPW)])
        plsc.subcore_barrier()

        @pl.loop(0, NCH, step=8)
        def _(j):
            for b in range(8):
                pltpu.make_async_copy(
                    onesv, acc.at[colv.at[j + b]], sem).start(add=True)
            for b in range(8):
                pltpu.make_async_copy(
                    onesv, acc.at[colv.at[j + b]], sem).wait()

        plsc.subcore_barrier()
        pltpu.sync_copy(acc.at[pl.ds(s * TPW, TPW)],
                        out_hbm.at[c].at[pl.ds(s * TPW, TPW)])

    return k(colh, ones128, zeros128)


def _spmm_partials(g, colg, rows, zeros128):
    @functools.partial(
        pl.kernel,
        out_type=jax.ShapeDtypeStruct((NC, N_ACC, D), jnp.float32),
        mesh=_MESH,
        scratch_types=[
            pltpu.VMEM((PNCH, CHUNK), jnp.int32),
            pltpu.VMEM((PNCH, CHUNK), jnp.int32),
            pltpu.VMEM((CHUNK, D), jnp.float32),
            pltpu.VMEM((CHUNK, D), jnp.float32),
            pltpu.VMEM((CHUNK, D), jnp.float32),
            pltpu.VMEM((CHUNK, D), jnp.float32),
            pltpu.VMEM_SHARED((N_ACC, D), jnp.float32),
            pltpu.SemaphoreType.DMA,
            pltpu.SemaphoreType.DMA,
            pltpu.SemaphoreType.DMA,
            pltpu.SemaphoreType.DMA,
            pltpu.SemaphoreType.DMA,
            pltpu.SemaphoreType.DMA,
            pltpu.SemaphoreType.DMA,
            pltpu.SemaphoreType.DMA,
        ],
    )
    def k(g_hbm, colg_hbm, rows_hbm, zeros_hbm, out_hbm, colv, rowv,
          d0, d1, d2, d3, acc, gs0, gs1, gs2, gs3, ss0, ss1, ss2, ss3):
        c = lax.axis_index("c")
        s = lax.axis_index("s")
        w = c * NS + s
        pltpu.sync_copy(zeros_hbm, acc.at[pl.ds(s * TPW, TPW)])

        bufs = (d0, d1, d2, d3)
        gsems = (gs0, gs1, gs2, gs3)
        ssems = (ss0, ss1, ss2, ss3)

        for ph in range(NPH):
            pltpu.sync_copy(colg_hbm.at[w].at[pl.ds(ph * PNCH, PNCH)], colv)
            pltpu.sync_copy(rows_hbm.at[w].at[pl.ds(ph * PNCH, PNCH)], rowv)
            if ph == 0:
                plsc.subcore_barrier()
            for b in range(NB):
                pltpu.make_async_copy(
                    g_hbm.at[colv.at[b]], bufs[b], gsems[b]).start()

            @pl.loop(0, PNCH, step=NB)
            def _(j):
                for b in range(NB):
                    pltpu.make_async_copy(
                        g_hbm.at[colv.at[j + b]], bufs[b], gsems[b]).wait()
                    pltpu.make_async_copy(
                        bufs[b], acc.at[rowv.at[j + b]],
                        ssems[b]).start(add=True)
                for b in range(NB):
                    pltpu.make_async_copy(
                        bufs[b], acc.at[rowv.at[j + b]], ssems[b]).wait()

                    @pl.when(j + NB + b < PNCH)
                    def _():
                        pltpu.make_async_copy(
                            g_hbm.at[colv.at[j + NB + b]], bufs[b],
                            gsems[b]).start()

        plsc.subcore_barrier()
        pltpu.sync_copy(acc.at[pl.ds(s * TPW, TPW)],
                        out_hbm.at[c].at[pl.ds(s * TPW, TPW)])

    return k(g, colg, rows, zeros128)


def _fc0(x, fc0_w, fc0_b):
    def body(x_ref, w_ref, b_ref, o_ref):
        o_ref[...] = jnp.maximum(
            jnp.dot(x_ref[...], w_ref[...],
                    preferred_element_type=jnp.float32,
                    precision=lax.Precision.HIGHEST) + b_ref[...], 0.0)

    return pl.pallas_call(
        body,
        grid=(N // RB,),
        in_specs=[
            pl.BlockSpec((RB, D), lambda i: (i, 0)),
            pl.BlockSpec((D, D), lambda i: (0, 0)),
            pl.BlockSpec((1, D), lambda i: (0, 0)),
        ],
        out_specs=pl.BlockSpec((RB, D), lambda i: (i, 0)),
        out_shape=jax.ShapeDtypeStruct((N, D), jnp.float32),
    )(x, fc0_w, fc0_b.reshape(1, D))


def _prep(degp, h):
    def body(degp_ref, h_ref, dis_ref, g_ref):
        d = degp_ref[...]
        deg = d[0][:, :1] + d[1][:, :1] + 1.0
        dis = lax.rsqrt(deg)
        dis_b = jnp.broadcast_to(dis, (RB, D))
        dis_ref[...] = dis_b
        g_ref[...] = dis_b * h_ref[...]

    return pl.pallas_call(
        body,
        grid=(N // RB,),
        in_specs=[
            pl.BlockSpec((NC, RB, D), lambda i: (0, i, 0)),
            pl.BlockSpec((RB, D), lambda i: (i, 0)),
        ],
        out_specs=[
            pl.BlockSpec((RB, D), lambda i: (i, 0)),
            pl.BlockSpec((RB, D), lambda i: (i, 0)),
        ],
        out_shape=[
            jax.ShapeDtypeStruct((N, D), jnp.float32),
            jax.ShapeDtypeStruct((N, D), jnp.float32),
        ],
    )(degp, h)


def _layer(sp, g, h0, dis, w, theta):
    def body(sp_ref, g_ref, h0_ref, dis_ref, w_ref, o_ref):
        s = sp_ref[...]
        dis_b = dis_ref[...]
        hi = dis_b * (s[0] + s[1] + g_ref[...])
        sup = (1.0 - ALPHA) * hi + ALPHA * h0_ref[...]
        hn = jnp.maximum(
            theta * jnp.dot(sup, w_ref[...],
                            preferred_element_type=jnp.float32,
                            precision=lax.Precision.HIGHEST)
            + (1.0 - theta) * sup, 0.0)
        o_ref[...] = dis_b * hn

    return pl.pallas_call(
        body,
        grid=(N // RB,),
        in_specs=[
            pl.BlockSpec((NC, RB, D), lambda i: (0, i, 0)),
            pl.BlockSpec((RB, D), lambda i: (i, 0)),
            pl.BlockSpec((RB, D), lambda i: (i, 0)),
            pl.BlockSpec((RB, D), lambda i: (i, 0)),
            pl.BlockSpec((D, D), lambda i: (0, 0)),
        ],
        out_specs=pl.BlockSpec((RB, D), lambda i: (i, 0)),
        out_shape=jax.ShapeDtypeStruct((N, D), jnp.float32),
    )(sp, g, h0, dis, w)


def _final(sp, g1, h0, dis, w2, fc1_w, fc1_b):
    def body(sp_ref, g_ref, h0_ref, dis_ref, w_ref, fw_ref, fb_ref, o_ref):
        s = sp_ref[...]
        hi = dis_ref[...] * (s[0] + s[1] + g_ref[...])
        sup = (1.0 - ALPHA) * hi + ALPHA * h0_ref[...]
        h2 = jnp.maximum(
            THETA2 * jnp.dot(sup, w_ref[...],
                             preferred_element_type=jnp.float32,
                             precision=lax.Precision.HIGHEST)
            + (1.0 - THETA2) * sup, 0.0)
        o = jnp.dot(h2, fw_ref[...],
                    preferred_element_type=jnp.float32,
                    precision=lax.Precision.HIGHEST) + fb_ref[...]
        m = jnp.max(o, axis=1, keepdims=True)
        o_ref[...] = o - m - jnp.log(
            jnp.sum(jnp.exp(o - m), axis=1, keepdims=True))

    return pl.pallas_call(
        body,
        grid=(N // RB,),
        in_specs=[
            pl.BlockSpec((NC, RB, D), lambda i: (0, i, 0)),
            pl.BlockSpec((RB, D), lambda i: (i, 0)),
            pl.BlockSpec((RB, D), lambda i: (i, 0)),
            pl.BlockSpec((RB, D), lambda i: (i, 0)),
            pl.BlockSpec((D, D), lambda i: (0, 0)),
            pl.BlockSpec((D, D), lambda i: (0, 0)),
            pl.BlockSpec((1, D), lambda i: (0, 0)),
        ],
        out_specs=pl.BlockSpec((RB, D), lambda i: (i, 0)),
        out_shape=jax.ShapeDtypeStruct((N, D), jnp.float32),
    )(sp, g1, h0, dis, w2, fc1_w, fc1_b.reshape(1, D))


def kernel(x, edge_index, fc0_w, fc0_b, w1, w2, fc1_w, fc1_b):
    row = edge_index[0]
    col = edge_index[1]
    pad = E_PAD - E
    colh = jnp.concatenate([col, jnp.full((pad,), N, jnp.int32)])
    colg = jnp.concatenate([col, jnp.zeros((pad,), jnp.int32)])
    rows = jnp.concatenate([row, jnp.full((pad,), N, jnp.int32)])
    colh = colh.reshape(NW, NCH, CHUNK)
    colg = colg.reshape(NW, NCH, CHUNK)
    rows = rows.reshape(NW, NCH, CHUNK)
    ones128 = jnp.ones((CHUNK, D), jnp.float32)
    zeros128 = jnp.zeros((TPW, D), jnp.float32)

    degp = _deg_partials(colh, ones128, zeros128)
    h0 = _fc0(x, fc0_w, fc0_b)
    dis, g0 = _prep(degp, h0)
    sp1 = _spmm_partials(g0, colg, rows, zeros128)
    g1 = _layer(sp1, g0, h0, dis, w1, THETA1)
    sp2 = _spmm_partials(g1, colg, rows, zeros128)
    return _final(sp2, g1, h0, dis, w2, fc1_w, fc1_b)


# speedup vs baseline: 24.7220x; 2.7740x over previous
import functools
import math

import jax
import jax.numpy as jnp
from jax import lax
from jax.experimental import pallas as pl
from jax.experimental.pallas import tpu as pltpu
from jax.experimental.pallas import tpu_sc as plsc

N = 10000
E = 320000
D = 128
NC = 2
NS = 16
NW = NC * NS
CHUNK = 64
NCH = 160
PNCH = 40
NPH = 4
NB = 4
E_PAD = NW * NCH * CHUNK
N_ACC = 10112
TPW = N_ACC // NS
RB = 2000

ALPHA = 0.1
THETA1 = math.log(2.0)
THETA2 = math.log(1.5)

_MESH = plsc.VectorSubcoreMesh(core_axis_name="c", subcore_axis_name="s")


def _deg_partials(colh, ones128, zeros128):
    @functools.partial(
        pl.kernel,
        out_type=jax.ShapeDtypeStruct((NC, N_ACC, D), jnp.float32),
        mesh=_MESH,
        scratch_types=[
            pltpu.VMEM((NCH, CHUNK), jnp.int32),
            pltpu.VMEM((CHUNK, D), jnp.float32),
            pltpu.VMEM_SHARED((N_ACC, D), jnp.float32),
            pltpu.SemaphoreType.DMA,
        ],
    )
    def k(colh_hbm, ones_hbm, zeros_hbm, out_hbm, colv, onesv, acc, sem):
        c = lax.axis_index("c")
        s = lax.axis_index("s")
        w = c * NS + s
        pltpu.sync_copy(colh_hbm.at[w], colv)
        pltpu.sync_copy(ones_hbm, onesv)
        pltpu.sync_copy(zeros_hbm, acc.at[pl.ds(s * TPW, TPW)])
        plsc.subcore_barrier()

        @pl.loop(0, NCH, step=8)
        def _(j):
            for b in range(8):
                pltpu.make_async_copy(
                    onesv, acc.at[colv.at[j + b]], sem).start(add=True)
            for b in range(8):
                pltpu.make_async_copy(
                    onesv, acc.at[colv.at[j + b]], sem).wait()

        plsc.subcore_barrier()
        pltpu.sync_copy(acc.at[pl.ds(s * TPW, TPW)],
                        out_hbm.at[c].at[pl.ds(s * TPW, TPW)])

    return k(colh, ones128, zeros128)


def _spmm_partials(g, colg, rows, zeros128):
    @functools.partial(
        pl.kernel,
        out_type=jax.ShapeDtypeStruct((NC, N_ACC, D), jnp.float32),
        mesh=_MESH,
        scratch_types=[
            pltpu.VMEM((PNCH, CHUNK), jnp.int32),
            pltpu.VMEM((PNCH, CHUNK), jnp.int32),
            pltpu.VMEM((CHUNK, D), jnp.float32),
            pltpu.VMEM((CHUNK, D), jnp.float32),
            pltpu.VMEM((CHUNK, D), jnp.float32),
            pltpu.VMEM((CHUNK, D), jnp.float32),
            pltpu.VMEM_SHARED((N_ACC, D), jnp.float32),
            pltpu.SemaphoreType.DMA,
            pltpu.SemaphoreType.DMA,
            pltpu.SemaphoreType.DMA,
            pltpu.SemaphoreType.DMA,
            pltpu.SemaphoreType.DMA,
            pltpu.SemaphoreType.DMA,
            pltpu.SemaphoreType.DMA,
            pltpu.SemaphoreType.DMA,
        ],
    )
    def k(g_hbm, colg_hbm, rows_hbm, zeros_hbm, out_hbm, colv, rowv,
          d0, d1, d2, d3, acc, gs0, gs1, gs2, gs3, ss0, ss1, ss2, ss3):
        c = lax.axis_index("c")
        s = lax.axis_index("s")
        w = c * NS + s
        pltpu.sync_copy(zeros_hbm, acc.at[pl.ds(s * TPW, TPW)])

        bufs = (d0, d1, d2, d3)
        gsems = (gs0, gs1, gs2, gs3)
        ssems = (ss0, ss1, ss2, ss3)

        for ph in range(NPH):
            pltpu.sync_copy(colg_hbm.at[w].at[pl.ds(ph * PNCH, PNCH)], colv)
            pltpu.sync_copy(rows_hbm.at[w].at[pl.ds(ph * PNCH, PNCH)], rowv)
            if ph == 0:
                plsc.subcore_barrier()
            for b in range(NB):
                pltpu.make_async_copy(
                    g_hbm.at[colv.at[b]], bufs[b], gsems[b]).start()

            @pl.loop(0, PNCH, step=NB)
            def _(j):
                for b in range(NB):
                    pltpu.make_async_copy(
                        g_hbm.at[colv.at[j + b]], bufs[b], gsems[b]).wait()
                    pltpu.make_async_copy(
                        bufs[b], acc.at[rowv.at[j + b]],
                        ssems[b]).start(add=True)
                for b in range(NB):
                    pltpu.make_async_copy(
                        bufs[b], acc.at[rowv.at[j + b]], ssems[b]).wait()

                    @pl.when(j + NB + b < PNCH)
                    def _():
                        pltpu.make_async_copy(
                            g_hbm.at[colv.at[j + NB + b]], bufs[b],
                            gsems[b]).start()

        plsc.subcore_barrier()
        pltpu.sync_copy(acc.at[pl.ds(s * TPW, TPW)],
                        out_hbm.at[c].at[pl.ds(s * TPW, TPW)])

    return k(g, colg, rows, zeros128)


def _fc0(x, fc0_w, fc0_b):
    def body(x_ref, w_ref, b_ref, o_ref):
        o_ref[...] = jnp.maximum(
            jnp.dot(x_ref[...], w_ref[...],
                    preferred_element_type=jnp.float32,
                    precision=lax.Precision.HIGHEST) + b_ref[...], 0.0)

    return pl.pallas_call(
        body,
        grid=(N // RB,),
        in_specs=[
            pl.BlockSpec((RB, D), lambda i: (i, 0)),
            pl.BlockSpec((D, D), lambda i: (0, 0)),
            pl.BlockSpec((1, D), lambda i: (0, 0)),
        ],
        out_specs=pl.BlockSpec((RB, D), lambda i: (i, 0)),
        out_shape=jax.ShapeDtypeStruct((N, D), jnp.float32),
    )(x, fc0_w, fc0_b.reshape(1, D))


def _prep(degp, h):
    def body(degp_ref, h_ref, dis_ref, g_ref):
        d = degp_ref[...]
        deg = d[0][:, :1] + d[1][:, :1] + 1.0
        dis = lax.rsqrt(deg)
        dis_b = jnp.broadcast_to(dis, (RB, D))
        dis_ref[...] = dis_b
        g_ref[...] = dis_b * h_ref[...]

    return pl.pallas_call(
        body,
        grid=(N // RB,),
        in_specs=[
            pl.BlockSpec((NC, RB, D), lambda i: (0, i, 0)),
            pl.BlockSpec((RB, D), lambda i: (i, 0)),
        ],
        out_specs=[
            pl.BlockSpec((RB, D), lambda i: (i, 0)),
            pl.BlockSpec((RB, D), lambda i: (i, 0)),
        ],
        out_shape=[
            jax.ShapeDtypeStruct((N, D), jnp.float32),
            jax.ShapeDtypeStruct((N, D), jnp.float32),
        ],
    )(degp, h)


def _layer(sp, g, h0, dis, w, theta):
    def body(sp_ref, g_ref, h0_ref, dis_ref, w_ref, o_ref):
        s = sp_ref[...]
        dis_b = dis_ref[...]
        hi = dis_b * (s[0] + s[1] + g_ref[...])
        sup = (1.0 - ALPHA) * hi + ALPHA * h0_ref[...]
        hn = jnp.maximum(
            theta * jnp.dot(sup, w_ref[...],
                            preferred_element_type=jnp.float32,
                            precision=lax.Precision.HIGHEST)
            + (1.0 - theta) * sup, 0.0)
        o_ref[...] = dis_b * hn

    return pl.pallas_call(
        body,
        grid=(N // RB,),
        in_specs=[
            pl.BlockSpec((NC, RB, D), lambda i: (0, i, 0)),
            pl.BlockSpec((RB, D), lambda i: (i, 0)),
            pl.BlockSpec((RB, D), lambda i: (i, 0)),
            pl.BlockSpec((RB, D), lambda i: (i, 0)),
            pl.BlockSpec((D, D), lambda i: (0, 0)),
        ],
        out_specs=pl.BlockSpec((RB, D), lambda i: (i, 0)),
        out_shape=jax.ShapeDtypeStruct((N, D), jnp.float32),
    )(sp, g, h0, dis, w)


def _final(sp, g1, h0, dis, w2, fc1_w, fc1_b):
    def body(sp_ref, g_ref, h0_ref, dis_ref, w_ref, fw_ref, fb_ref, o_ref):
        s = sp_ref[...]
        hi = dis_ref[...] * (s[0] + s[1] + g_ref[...])
        sup = (1.0 - ALPHA) * hi + ALPHA * h0_ref[...]
        h2 = jnp.maximum(
            THETA2 * jnp.dot(sup, w_ref[...],
                             preferred_element_type=jnp.float32,
                             precision=lax.Precision.HIGHEST)
            + (1.0 - THETA2) * sup, 0.0)
        o = jnp.dot(h2, fw_ref[...],
                    preferred_element_type=jnp.float32,
                    precision=lax.Precision.HIGHEST) + fb_ref[...]
        m = jnp.max(o, axis=1, keepdims=True)
        o_ref[...] = o - m - jnp.log(
            jnp.sum(jnp.exp(o - m), axis=1, keepdims=True))

    return pl.pallas_call(
        body,
        grid=(N // RB,),
        in_specs=[
            pl.BlockSpec((NC, RB, D), lambda i: (0, i, 0)),
            pl.BlockSpec((RB, D), lambda i: (i, 0)),
            pl.BlockSpec((RB, D), lambda i: (i, 0)),
            pl.BlockSpec((RB, D), lambda i: (i, 0)),
            pl.BlockSpec((D, D), lambda i: (0, 0)),
            pl.BlockSpec((D, D), lambda i: (0, 0)),
            pl.BlockSpec((1, D), lambda i: (0, 0)),
        ],
        out_specs=pl.BlockSpec((RB, D), lambda i: (i, 0)),
        out_shape=jax.ShapeDtypeStruct((N, D), jnp.float32),
    )(sp, g1, h0, dis, w2, fc1_w, fc1_b.reshape(1, D))


def kernel(x, edge_index, fc0_w, fc0_b, w1, w2, fc1_w, fc1_b):
    row = edge_index[0]
    col = edge_index[1]
    padw = (E_PAD - E) // NW
    pad_h = jnp.full((NW, padw), N, jnp.int32)
    pad_g = jnp.broadcast_to(jnp.arange(padw, dtype=jnp.int32), (NW, padw))
    colh = jnp.concatenate([col.reshape(NW, E // NW), pad_h], axis=1)
    colg = jnp.concatenate([col.reshape(NW, E // NW), pad_g], axis=1)
    rows = jnp.concatenate([row.reshape(NW, E // NW), pad_h], axis=1)
    colh = colh.reshape(NW, NCH, CHUNK)
    colg = colg.reshape(NW, NCH, CHUNK)
    rows = rows.reshape(NW, NCH, CHUNK)
    ones128 = jnp.ones((CHUNK, D), jnp.float32)
    zeros128 = jnp.zeros((TPW, D), jnp.float32)

    degp = _deg_partials(colh, ones128, zeros128)
    h0 = _fc0(x, fc0_w, fc0_b)
    dis, g0 = _prep(degp, h0)
    sp1 = _spmm_partials(g0, colg, rows, zeros128)
    g1 = _layer(sp1, g0, h0, dis, w1, THETA1)
    sp2 = _spmm_partials(g1, colg, rows, zeros128)
    return _final(sp2, g1, h0, dis, w2, fc1_w, fc1_b)


# speedup vs baseline: 28.4005x; 1.1488x over previous
import dataclasses
import functools
import math

import jax
import jax.numpy as jnp
from jax import lax
from jax.experimental import pallas as pl
from jax.experimental.pallas import tpu as pltpu
from jax.experimental.pallas import tpu_sc as plsc

N = 10000
E = 320000
D = 128
NC = 2
NS = 16
NW = NC * NS
CHUNK = 64
NCH = 160
PNCH = 40
NPH = 4
NB = 4
E_PAD = NW * NCH * CHUNK
PADT = E_PAD - E
N_ACC = 10112
TPW = N_ACC // NS
N_DEG = 10240
TPD = N_DEG // NS
RB = 2000

ALPHA = 0.1
THETA1 = math.log(2.0)
THETA2 = math.log(1.5)

_MESH = plsc.VectorSubcoreMesh(core_axis_name="c", subcore_axis_name="s")

_CP = pltpu.CompilerParams()
if "needs_layout_passes" in pltpu.CompilerParams.__dataclass_fields__:
    _CP = dataclasses.replace(_CP, needs_layout_passes=False)


def _deg_partials(colh):
    @functools.partial(
        pl.kernel,
        out_type=jax.ShapeDtypeStruct((NC, N_DEG, D), jnp.float32),
        mesh=_MESH,
        compiler_params=_CP,
        scratch_types=[
            pltpu.VMEM((NCH, CHUNK), jnp.int32),
            pltpu.VMEM((N_DEG,), jnp.float32),
            pltpu.VMEM((NS, TPD), jnp.float32),
            pltpu.VMEM((TPD,), jnp.float32),
            pltpu.VMEM((TPD // 2, D), jnp.float32),
            pltpu.VMEM_SHARED((NS, N_DEG), jnp.float32),
        ],
    )
    def k(colh_hbm, out_hbm, colv, hist, redbuf, tot, outbuf, shist):
        c = lax.axis_index("c")
        s = lax.axis_index("s")
        w = c * NS + s
        pltpu.sync_copy(colh_hbm.at[w], colv)

        @pl.loop(0, N_DEG // 16)
        def _(i):
            hist[pl.ds(i * 16, 16)] = jnp.zeros((16,), jnp.float32)

        ones = jnp.ones((16,), jnp.float32)

        @pl.loop(0, NCH)
        def _(j):
            for k in range(CHUNK // 16):
                idx = colv[j, pl.ds(k * 16, 16)]
                plsc.addupdate_scatter(hist, [idx], ones)

        pltpu.sync_copy(hist, shist.at[s])
        plsc.subcore_barrier()
        for t in range(NS):
            pltpu.sync_copy(shist.at[t].at[pl.ds(s * TPD, TPD)],
                            redbuf.at[t])

        @pl.loop(0, TPD // 16)
        def _(i):
            v = redbuf[0, pl.ds(i * 16, 16)]
            for t in range(1, NS):
                v = v + redbuf[t, pl.ds(i * 16, 16)]
            tot[pl.ds(i * 16, 16)] = v

        for h in range(2):
            @pl.loop(0, TPD // 32)
            def _(i):
                v = tot[pl.ds(h * (TPD // 2) + i * 16, 16)]
                for k in range(16):
                    outbuf[i * 16 + k, pl.ds(0, 16)] = jnp.full((16,), v[k])

            pltpu.sync_copy(
                outbuf,
                out_hbm.at[c].at[pl.ds(s * TPD + h * (TPD // 2), TPD // 2)])

    return k(colh)


def _spmm_partials(g, colg, rows, zeros128):
    @functools.partial(
        pl.kernel,
        out_type=jax.ShapeDtypeStruct((NC, N_ACC, D), jnp.float32),
        mesh=_MESH,
        scratch_types=[
            pltpu.VMEM((PNCH, CHUNK), jnp.int32),
            pltpu.VMEM((PNCH, CHUNK), jnp.int32),
            pltpu.VMEM((CHUNK, D), jnp.float32),
            pltpu.VMEM((CHUNK, D), jnp.float32),
            pltpu.VMEM((CHUNK, D), jnp.float32),
            pltpu.VMEM((CHUNK, D), jnp.float32),
            pltpu.VMEM_SHARED((N_ACC, D), jnp.float32),
            pltpu.SemaphoreType.DMA,
            pltpu.SemaphoreType.DMA,
            pltpu.SemaphoreType.DMA,
            pltpu.SemaphoreType.DMA,
            pltpu.SemaphoreType.DMA,
            pltpu.SemaphoreType.DMA,
            pltpu.SemaphoreType.DMA,
            pltpu.SemaphoreType.DMA,
        ],
    )
    def k(g_hbm, colg_hbm, rows_hbm, zeros_hbm, out_hbm, colv, rowv,
          d0, d1, d2, d3, acc,
          gs0, gs1, gs2, gs3, ss0, ss1, ss2, ss3):
        c = lax.axis_index("c")
        s = lax.axis_index("s")
        w = c * NS + s
        pltpu.sync_copy(zeros_hbm, acc.at[pl.ds(s * TPW, TPW)])

        bufs = (d0, d1, d2, d3)
        gsems = (gs0, gs1, gs2, gs3)
        ssems = (ss0, ss1, ss2, ss3)

        for ph in range(NPH):
            pltpu.sync_copy(colg_hbm.at[w].at[pl.ds(ph * PNCH, PNCH)], colv)
            pltpu.sync_copy(rows_hbm.at[w].at[pl.ds(ph * PNCH, PNCH)], rowv)
            if ph == 0:
                plsc.subcore_barrier()
            for b in range(NB):
                pltpu.make_async_copy(
                    g_hbm.at[colv.at[b]], bufs[b], gsems[b]).start()

            @pl.loop(0, PNCH, step=NB)
            def _(j):
                for b in range(NB):
                    pltpu.make_async_copy(
                        g_hbm.at[colv.at[j + b]], bufs[b], gsems[b]).wait()
                    pltpu.make_async_copy(
                        bufs[b], acc.at[rowv.at[j + b]],
                        ssems[b]).start(add=True)
                for b in range(NB):
                    pltpu.make_async_copy(
                        bufs[b], acc.at[rowv.at[j + b]], ssems[b]).wait()

                    @pl.when(j + NB + b < PNCH)
                    def _():
                        pltpu.make_async_copy(
                            g_hbm.at[colv.at[j + NB + b]], bufs[b],
                            gsems[b]).start()

        plsc.subcore_barrier()
        pltpu.sync_copy(acc.at[pl.ds(s * TPW, TPW)],
                        out_hbm.at[c].at[pl.ds(s * TPW, TPW)])

    return k(g, colg, rows, zeros128)


def _fc0(x, fc0_w, fc0_b):
    def body(x_ref, w_ref, b_ref, o_ref):
        o_ref[...] = jnp.maximum(
            jnp.dot(x_ref[...], w_ref[...],
                    preferred_element_type=jnp.float32,
                    precision=lax.Precision.HIGHEST) + b_ref[...], 0.0)

    return pl.pallas_call(
        body,
        grid=(N // RB,),
        in_specs=[
            pl.BlockSpec((RB, D), lambda i: (i, 0)),
            pl.BlockSpec((D, D), lambda i: (0, 0)),
            pl.BlockSpec((1, D), lambda i: (0, 0)),
        ],
        out_specs=pl.BlockSpec((RB, D), lambda i: (i, 0)),
        out_shape=jax.ShapeDtypeStruct((N, D), jnp.float32),
    )(x, fc0_w, fc0_b.reshape(1, D))


def _prep(degp, h):
    def body(degp_ref, h_ref, dis_ref, g_ref):
        d = degp_ref[...]
        ids = (pl.program_id(0) * RB
               + lax.broadcasted_iota(jnp.int32, (RB, 1), 0))
        corr = jnp.where(ids < PADT, 1.0, 0.0)
        deg = d[0][:, :1] + d[1][:, :1] + 1.0 - corr
        dis = lax.rsqrt(deg)
        dis_b = jnp.broadcast_to(dis, (RB, D))
        dis_ref[...] = dis_b
        g_ref[...] = dis_b * h_ref[...]

    return pl.pallas_call(
        body,
        grid=(N // RB,),
        in_specs=[
            pl.BlockSpec((NC, RB, D), lambda i: (0, i, 0)),
            pl.BlockSpec((RB, D), lambda i: (i, 0)),
        ],
        out_specs=[
            pl.BlockSpec((RB, D), lambda i: (i, 0)),
            pl.BlockSpec((RB, D), lambda i: (i, 0)),
        ],
        out_shape=[
            jax.ShapeDtypeStruct((N, D), jnp.float32),
            jax.ShapeDtypeStruct((N, D), jnp.float32),
        ],
    )(degp, h)


def _layer(sp, g, h0, dis, w, theta):
    def body(sp_ref, g_ref, h0_ref, dis_ref, w_ref, o_ref):
        s = sp_ref[...]
        dis_b = dis_ref[...]
        hi = dis_b * (s[0] + s[1] + g_ref[...])
        sup = (1.0 - ALPHA) * hi + ALPHA * h0_ref[...]
        hn = jnp.maximum(
            theta * jnp.dot(sup, w_ref[...],
                            preferred_element_type=jnp.float32,
                            precision=lax.Precision.HIGHEST)
            + (1.0 - theta) * sup, 0.0)
        o_ref[...] = dis_b * hn

    return pl.pallas_call(
        body,
        grid=(N // RB,),
        in_specs=[
            pl.BlockSpec((NC, RB, D), lambda i: (0, i, 0)),
            pl.BlockSpec((RB, D), lambda i: (i, 0)),
            pl.BlockSpec((RB, D), lambda i: (i, 0)),
            pl.BlockSpec((RB, D), lambda i: (i, 0)),
            pl.BlockSpec((D, D), lambda i: (0, 0)),
        ],
        out_specs=pl.BlockSpec((RB, D), lambda i: (i, 0)),
        out_shape=jax.ShapeDtypeStruct((N, D), jnp.float32),
    )(sp, g, h0, dis, w)


def _final(sp, g1, h0, dis, w2, fc1_w, fc1_b):
    def body(sp_ref, g_ref, h0_ref, dis_ref, w_ref, fw_ref, fb_ref, o_ref):
        s = sp_ref[...]
        hi = dis_ref[...] * (s[0] + s[1] + g_ref[...])
        sup = (1.0 - ALPHA) * hi + ALPHA * h0_ref[...]
        h2 = jnp.maximum(
            THETA2 * jnp.dot(sup, w_ref[...],
                             preferred_element_type=jnp.float32,
                             precision=lax.Precision.HIGHEST)
            + (1.0 - THETA2) * sup, 0.0)
        o = jnp.dot(h2, fw_ref[...],
                    preferred_element_type=jnp.float32,
                    precision=lax.Precision.HIGHEST) + fb_ref[...]
        m = jnp.max(o, axis=1, keepdims=True)
        o_ref[...] = o - m - jnp.log(
            jnp.sum(jnp.exp(o - m), axis=1, keepdims=True))

    return pl.pallas_call(
        body,
        grid=(N // RB,),
        in_specs=[
            pl.BlockSpec((NC, RB, D), lambda i: (0, i, 0)),
            pl.BlockSpec((RB, D), lambda i: (i, 0)),
            pl.BlockSpec((RB, D), lambda i: (i, 0)),
            pl.BlockSpec((RB, D), lambda i: (i, 0)),
            pl.BlockSpec((D, D), lambda i: (0, 0)),
            pl.BlockSpec((D, D), lambda i: (0, 0)),
            pl.BlockSpec((1, D), lambda i: (0, 0)),
        ],
        out_specs=pl.BlockSpec((RB, D), lambda i: (i, 0)),
        out_shape=jax.ShapeDtypeStruct((N, D), jnp.float32),
    )(sp, g1, h0, dis, w2, fc1_w, fc1_b.reshape(1, D))


def kernel(x, edge_index, fc0_w, fc0_b, w1, w2, fc1_w, fc1_b):
    ar = jnp.arange(PADT, dtype=jnp.int32)
    pad_rc = jnp.stack([N + ar % (N_ACC - N), ar])
    eip = jnp.concatenate([edge_index, pad_rc], axis=1)
    rows = eip[0].reshape(NW, NCH, CHUNK)
    cols = eip[1].reshape(NW, NCH, CHUNK)
    zeros128 = jnp.zeros((TPW, D), jnp.float32)

    degp = _deg_partials(cols)
    h0 = _fc0(x, fc0_w, fc0_b)
    dis, g0 = _prep(degp, h0)
    sp1 = _spmm_partials(g0, cols, rows, zeros128)
    g1 = _layer(sp1, g0, h0, dis, w1, THETA1)
    sp2 = _spmm_partials(g1, cols, rows, zeros128)
    return _final(sp2, g1, h0, dis, w2, fc1_w, fc1_b)
